# baseline (speedup 1.0000x reference)
import functools

import jax
import jax.numpy as jnp
from jax import lax
from jax.experimental import pallas as pl
from jax.experimental.pallas import tpu as pltpu
from jax.experimental.pallas import tpu_sc as plsc

N = 10000
E = 320000
D = 128
NMOL = 512

NC = 2
NS = 16
NW = NC * NS
EPW = E // NW
CHUNK = 120
NCHUNK = 84
EPW_PAD = NCHUNK * CHUNK
NBUF = 3
NGRP = NCHUNK // NBUF
HPAD = 8
ACC_ROWS = N
ZROWS = 624
TAIL0 = ZROWS * NS
TAILR = N - TAIL0

_SC_MESH = plsc.VectorSubcoreMesh(core_axis_name="c", subcore_axis_name="s")


@functools.partial(
    pl.kernel,
    out_type=jax.ShapeDtypeStruct((NC, N, D), jnp.float32),
    mesh=_SC_MESH,
    scratch_types=[
        pltpu.VMEM_SHARED((ACC_ROWS, D), jnp.float32),
        pltpu.VMEM((2, NBUF, 2, CHUNK), jnp.int32),
        pltpu.VMEM((NBUF, CHUNK, D), jnp.float32),
        pltpu.SemaphoreType.DMA((NBUF,)),
        pltpu.SemaphoreType.DMA((NBUF,)),
        pltpu.SemaphoreType.DMA((2,)),
    ],
)
def _spmm(h_hbm, idx_hbm, out_hbm, acc_sh, idx_v, rows_v, gsem, ssem,
          isem):
    c = lax.axis_index("c")
    s = lax.axis_index("s")
    wid = s * NC + c

    def _idx_start(g, slot):
        pltpu.async_copy(idx_hbm.at[wid, pl.ds(g * NBUF, NBUF)],
                         idx_v.at[slot], isem.at[slot])

    def _idx_wait(g, slot):
        pltpu.make_async_copy(idx_hbm.at[wid, pl.ds(g * NBUF, NBUF)],
                              idx_v.at[slot], isem.at[slot]).wait()

    def _gather_start(slot, b):
        pltpu.async_copy(h_hbm.at[idx_v.at[slot, b, 0]], rows_v.at[b],
                         gsem.at[b])

    def _scatter_start(slot, b):
        pltpu.async_copy(rows_v.at[b], acc_sh.at[idx_v.at[slot, b, 1]],
                        ssem.at[b], add=True)

    def _drain(sem, b):
        pltpu.make_async_copy(h_hbm.at[pl.ds(0, CHUNK)], rows_v.at[b],
                              sem.at[b]).wait()

    _idx_start(0, 0)
    _idx_start(1, 1)

    z16 = jnp.zeros((16,), jnp.float32)
    for r in range(16):
        for j in range(D // 16):
            rows_v[0, r, pl.ds(j * 16, 16)] = z16
    zero_v = rows_v.at[0, pl.ds(0, 16)]
    row0 = s * ZROWS

    def _zero_step(k, carry):
        pltpu.sync_copy(zero_v, acc_sh.at[pl.ds(row0 + k * 16, 16)])
        return carry

    lax.fori_loop(0, ZROWS // 16, _zero_step, 0)

    @pl.when(s == NS - 1)
    def _zero_tail():
        pltpu.sync_copy(zero_v, acc_sh.at[pl.ds(TAIL0, TAILR)])

    plsc.subcore_barrier()

    _idx_wait(0, 0)
    for b in range(NBUF):
        _gather_start(0, b)

    def _one_group(g, slot):
        nslot = 1 - slot

        @pl.when(g + 1 < NGRP)
        def _wait_next_idx():
            _idx_wait(g + 1, nslot)

        for b in range(NBUF):
            _drain(gsem, b)
            _scatter_start(slot, b)

        for b in range(NBUF):
            _drain(ssem, b)

            @pl.when(g + 1 < NGRP)
            def _refill():
                _gather_start(nslot, b)

        @pl.when(g + 2 < NGRP)
        def _prefetch_idx():
            _idx_start(g + 2, slot)

    def _pair(it, carry):
        _one_group(2 * it, 0)
        _one_group(2 * it + 1, 1)
        return carry

    lax.fori_loop(0, NGRP // 2, _pair, 0)
    plsc.subcore_barrier()

    pltpu.sync_copy(acc_sh.at[pl.ds(row0, ZROWS)],
                    out_hbm.at[c, pl.ds(row0, ZROWS)])

    @pl.when(s == NS - 1)
    def _copy_tail():
        pltpu.sync_copy(acc_sh.at[pl.ds(TAIL0, TAILR)],
                        out_hbm.at[c, pl.ds(TAIL0, TAILR)])


def _update_body(p_ref, x_ref, wm_ref, wu_ref, o_ref):
    g = p_ref[0] + p_ref[1]
    t = jnp.dot(g, wm_ref[...], preferred_element_type=jnp.float32)
    t = jnp.dot(t, wu_ref[...], preferred_element_type=jnp.float32)
    h = jnp.maximum(t + x_ref[...], 0.0)
    o_ref[...] = jnp.concatenate([h, jnp.zeros((HPAD, D), jnp.float32)], 0)


_update = pl.pallas_call(
    _update_body,
    out_shape=jax.ShapeDtypeStruct((N + HPAD, D), jnp.float32),
)

_POOL_CHUNK = 1000


def _final_body(p_ref, x_ref, ids_ref, wm_ref, wu_ref, w1_ref, b1_ref,
                w2_ref, b2_ref, wc1_ref, bc1_ref, wc2_ref, bc2_ref, o_ref):
    g = p_ref[0] + p_ref[1]
    t = jnp.dot(g, wm_ref[...], preferred_element_type=jnp.float32)
    t = jnp.dot(t, wu_ref[...], preferred_element_type=jnp.float32)
    h = jnp.maximum(t + x_ref[...], 0.0)

    pool = jnp.zeros((NMOL, D), jnp.float32)
    counts = jnp.zeros((NMOL, 1), jnp.float32)
    ids = ids_ref[...]
    for ci in range(N // _POOL_CHUNK):
        sl = slice(ci * _POOL_CHUNK, (ci + 1) * _POOL_CHUNK)
        onehot = (lax.broadcasted_iota(jnp.int32, (NMOL, _POOL_CHUNK), 0)
                  == ids[:, sl]).astype(jnp.float32)
        pool = pool + jnp.dot(onehot, h[sl, :],
                              preferred_element_type=jnp.float32)
        counts = counts + jnp.sum(onehot, axis=1, keepdims=True)
    mol = pool / jnp.maximum(counts, 1.0)

    t1 = jnp.maximum(jnp.dot(mol, w1_ref[...],
                             preferred_element_type=jnp.float32)
                     + b1_ref[...], 0.0)
    og = jnp.dot(t1, w2_ref[...], preferred_element_type=jnp.float32) \
        + b2_ref[...]
    t2 = jnp.maximum(jnp.dot(og, wc1_ref[...],
                             preferred_element_type=jnp.float32)
                     + bc1_ref[...], 0.0)
    z = jnp.dot(t2, wc2_ref[...], preferred_element_type=jnp.float32) \
        + bc2_ref[...]
    o_ref[...] = jax.nn.sigmoid(z)


_final = pl.pallas_call(
    _final_body,
    out_shape=jax.ShapeDtypeStruct((NMOL, 1), jnp.float32),
)


def kernel(x, edge_index, mol_ids, depth, W_msg, W_upd, W_ffn1, b_ffn1,
           W_ffn2, b_ffn2, W_cls1, b_cls1, W_cls2, b_cls2):
    src = edge_index[0].reshape(NW, EPW)
    dst = edge_index[1].reshape(NW, EPW)
    pad = EPW_PAD - EPW
    src3 = jnp.pad(src, ((0, 0), (0, pad)),
                   constant_values=N).reshape(NW, NCHUNK, 1, CHUNK)
    dst_pad = (jnp.arange(pad, dtype=jnp.int32)[None, :] * 97
               + jnp.arange(NW, dtype=jnp.int32)[:, None] * 331) % N
    dst3 = jnp.concatenate([dst, dst_pad], axis=1).reshape(
        NW, NCHUNK, 1, CHUNK)
    idx4 = jnp.concatenate([src3, dst3], axis=2)
    x_ext = jnp.pad(x, ((0, HPAD), (0, 0)))
    ids2d = mol_ids.reshape(1, N)
    b1 = b_ffn1.reshape(1, -1)
    b2 = b_ffn2.reshape(1, -1)
    bc1 = b_cls1.reshape(1, -1)
    bc2 = b_cls2.reshape(1, -1)

    h = x_ext
    for _ in range(2):
        p = _spmm(h, idx4)
        h = _update(p, x, W_msg, W_upd)
    p = _spmm(h, idx4)
    return _final(p, x, ids2d, W_msg, W_upd, W_ffn1, b1, W_ffn2, b2,
                  W_cls1, bc1, W_cls2, bc2)


# speedup vs baseline: 1.0068x; 1.0068x over previous
import functools

import jax
import jax.numpy as jnp
from jax import lax
from jax.experimental import pallas as pl
from jax.experimental.pallas import tpu as pltpu
from jax.experimental.pallas import tpu_sc as plsc

N = 10000
E = 320000
D = 128
NMOL = 512

NC = 2
NS = 16
NW = NC * NS
EPW = E // NW
CHUNK = 120
NCHUNK = 84
EPW_PAD = NCHUNK * CHUNK
NBUF = 3
NGRP = NCHUNK // NBUF
HPAD = 8
ACC_ROWS = N
ZROWS = 624
TAIL0 = ZROWS * NS
TAILR = N - TAIL0

_SC_MESH = plsc.VectorSubcoreMesh(core_axis_name="c", subcore_axis_name="s")


@functools.partial(
    pl.kernel,
    out_type=jax.ShapeDtypeStruct((NC, N, D), jnp.float32),
    mesh=_SC_MESH,
    scratch_types=[
        pltpu.VMEM_SHARED((ACC_ROWS, D), jnp.float32),
        pltpu.VMEM((2, NBUF, 2, CHUNK), jnp.int32),
        pltpu.VMEM((NBUF, CHUNK, D), jnp.float32),
        pltpu.SemaphoreType.DMA((NBUF,)),
        pltpu.SemaphoreType.DMA((NBUF,)),
        pltpu.SemaphoreType.DMA((2,)),
        pltpu.SemaphoreType.DMA,
    ],
)
def _spmm(h_hbm, idx_hbm, out_hbm, acc_sh, idx_v, rows_v, gsem, ssem,
          isem, zsem):
    c = lax.axis_index("c")
    s = lax.axis_index("s")
    wid = s * NC + c

    def _idx_start(g, slot):
        pltpu.async_copy(idx_hbm.at[wid, pl.ds(g * NBUF, NBUF)],
                         idx_v.at[slot], isem.at[slot])

    def _idx_wait(g, slot):
        pltpu.make_async_copy(idx_hbm.at[wid, pl.ds(g * NBUF, NBUF)],
                              idx_v.at[slot], isem.at[slot]).wait()

    def _gather_start(slot, b):
        pltpu.async_copy(h_hbm.at[idx_v.at[slot, b, 0]], rows_v.at[b],
                         gsem.at[b])

    def _scatter_start(slot, b):
        pltpu.async_copy(rows_v.at[b], acc_sh.at[idx_v.at[slot, b, 1]],
                        ssem.at[b], add=True)

    def _drain(sem, b):
        pltpu.make_async_copy(h_hbm.at[pl.ds(0, CHUNK)], rows_v.at[b],
                              sem.at[b]).wait()

    _idx_start(0, 0)
    _idx_start(1, 1)

    z16 = jnp.zeros((16,), jnp.float32)
    for r in range(CHUNK):
        for j in range(D // 16):
            rows_v[0, r, pl.ds(j * 16, 16)] = z16
    zero_v = rows_v.at[0]
    row0 = s * ZROWS
    nfull = ZROWS // CHUNK
    rem = ZROWS - nfull * CHUNK
    for k in range(nfull):
        pltpu.async_copy(zero_v, acc_sh.at[pl.ds(row0 + k * CHUNK, CHUNK)],
                         zsem)
    pltpu.async_copy(zero_v.at[pl.ds(0, rem)],
                     acc_sh.at[pl.ds(row0 + nfull * CHUNK, rem)], zsem)

    @pl.when(s == NS - 1)
    def _zero_tail():
        pltpu.async_copy(zero_v.at[pl.ds(0, TAILR)],
                         acc_sh.at[pl.ds(TAIL0, TAILR)], zsem)

    for k in range(nfull):
        pltpu.make_async_copy(zero_v, acc_sh.at[pl.ds(row0, CHUNK)],
                              zsem).wait()
    pltpu.make_async_copy(zero_v.at[pl.ds(0, rem)],
                          acc_sh.at[pl.ds(row0, rem)], zsem).wait()

    @pl.when(s == NS - 1)
    def _zero_tail_wait():
        pltpu.make_async_copy(zero_v.at[pl.ds(0, TAILR)],
                              acc_sh.at[pl.ds(TAIL0, TAILR)], zsem).wait()

    plsc.subcore_barrier()

    _idx_wait(0, 0)
    for b in range(NBUF):
        _gather_start(0, b)

    def _one_group(g, slot):
        nslot = 1 - slot

        @pl.when(g + 1 < NGRP)
        def _wait_next_idx():
            _idx_wait(g + 1, nslot)

        for b in range(NBUF):
            _drain(gsem, b)
            _scatter_start(slot, b)

        for b in range(NBUF):
            _drain(ssem, b)

            @pl.when(g + 1 < NGRP)
            def _refill():
                _gather_start(nslot, b)

        @pl.when(g + 2 < NGRP)
        def _prefetch_idx():
            _idx_start(g + 2, slot)

    def _pair(it, carry):
        _one_group(2 * it, 0)
        _one_group(2 * it + 1, 1)
        return carry

    lax.fori_loop(0, NGRP // 2, _pair, 0)
    plsc.subcore_barrier()

    pltpu.sync_copy(acc_sh.at[pl.ds(row0, ZROWS)],
                    out_hbm.at[c, pl.ds(row0, ZROWS)])

    @pl.when(s == NS - 1)
    def _copy_tail():
        pltpu.sync_copy(acc_sh.at[pl.ds(TAIL0, TAILR)],
                        out_hbm.at[c, pl.ds(TAIL0, TAILR)])


def _update_body(p_ref, x_ref, wm_ref, wu_ref, o_ref):
    g = p_ref[0] + p_ref[1]
    t = jnp.dot(g, wm_ref[...], preferred_element_type=jnp.float32)
    t = jnp.dot(t, wu_ref[...], preferred_element_type=jnp.float32)
    h = jnp.maximum(t + x_ref[...], 0.0)
    o_ref[...] = jnp.concatenate([h, jnp.zeros((HPAD, D), jnp.float32)], 0)


_update = pl.pallas_call(
    _update_body,
    out_shape=jax.ShapeDtypeStruct((N + HPAD, D), jnp.float32),
)

_POOL_CHUNK = 1000


def _final_body(p_ref, x_ref, ids_ref, wm_ref, wu_ref, w1_ref, b1_ref,
                w2_ref, b2_ref, wc1_ref, bc1_ref, wc2_ref, bc2_ref, o_ref):
    g = p_ref[0] + p_ref[1]
    t = jnp.dot(g, wm_ref[...], preferred_element_type=jnp.float32)
    t = jnp.dot(t, wu_ref[...], preferred_element_type=jnp.float32)
    h = jnp.maximum(t + x_ref[...], 0.0)

    pool = jnp.zeros((NMOL, D), jnp.float32)
    counts = jnp.zeros((NMOL, 1), jnp.float32)
    ids = ids_ref[...]
    for ci in range(N // _POOL_CHUNK):
        sl = slice(ci * _POOL_CHUNK, (ci + 1) * _POOL_CHUNK)
        onehot = (lax.broadcasted_iota(jnp.int32, (NMOL, _POOL_CHUNK), 0)
                  == ids[:, sl]).astype(jnp.float32)
        pool = pool + jnp.dot(onehot, h[sl, :],
                              preferred_element_type=jnp.float32)
        counts = counts + jnp.sum(onehot, axis=1, keepdims=True)
    mol = pool / jnp.maximum(counts, 1.0)

    t1 = jnp.maximum(jnp.dot(mol, w1_ref[...],
                             preferred_element_type=jnp.float32)
                     + b1_ref[...], 0.0)
    og = jnp.dot(t1, w2_ref[...], preferred_element_type=jnp.float32) \
        + b2_ref[...]
    t2 = jnp.maximum(jnp.dot(og, wc1_ref[...],
                             preferred_element_type=jnp.float32)
                     + bc1_ref[...], 0.0)
    z = jnp.dot(t2, wc2_ref[...], preferred_element_type=jnp.float32) \
        + bc2_ref[...]
    o_ref[...] = jax.nn.sigmoid(z)


_final = pl.pallas_call(
    _final_body,
    out_shape=jax.ShapeDtypeStruct((NMOL, 1), jnp.float32),
)


def kernel(x, edge_index, mol_ids, depth, W_msg, W_upd, W_ffn1, b_ffn1,
           W_ffn2, b_ffn2, W_cls1, b_cls1, W_cls2, b_cls2):
    src = edge_index[0].reshape(NW, EPW)
    dst = edge_index[1].reshape(NW, EPW)
    pad = EPW_PAD - EPW
    src3 = jnp.pad(src, ((0, 0), (0, pad)),
                   constant_values=N).reshape(NW, NCHUNK, 1, CHUNK)
    dst_pad = (jnp.arange(pad, dtype=jnp.int32)[None, :] * 97
               + jnp.arange(NW, dtype=jnp.int32)[:, None] * 331) % N
    dst3 = jnp.concatenate([dst, dst_pad], axis=1).reshape(
        NW, NCHUNK, 1, CHUNK)
    idx4 = jnp.concatenate([src3, dst3], axis=2)
    x_ext = jnp.pad(x, ((0, HPAD), (0, 0)))
    ids2d = mol_ids.reshape(1, N)
    b1 = b_ffn1.reshape(1, -1)
    b2 = b_ffn2.reshape(1, -1)
    bc1 = b_cls1.reshape(1, -1)
    bc2 = b_cls2.reshape(1, -1)

    h = x_ext
    for _ in range(2):
        p = _spmm(h, idx4)
        h = _update(p, x, W_msg, W_upd)
    p = _spmm(h, idx4)
    return _final(p, x, ids2d, W_msg, W_upd, W_ffn1, b1, W_ffn2, b2,
                  W_cls1, bc1, W_cls2, bc2)
